# baseline (speedup 1.0000x reference)
import functools

import jax
import jax.numpy as jnp
from jax import lax
from jax.experimental import pallas as pl
from jax.experimental.pallas import tpu as pltpu
from jax.experimental.pallas import tpu_sc as plsc

_NUM_STEPS = 2
_K = 10
_N = 10000
_E = 160000
_D = 256
_RD = 32
_NP = 10240
_BM = 256
_NEG = -1e30

_NC, _NS = 2, 16
_CH1 = 80
_CH2 = 40
_EP = _NS * _CH1 * 128
_NR = 10112
_RT = _NR // _NS


def _simtopk_body(hs_ref, ht_ref, vals_ref, idx_ref, strip_ref):
    scores = jax.lax.dot_general(
        hs_ref[...], ht_ref[...],
        (((1,), (1,)), ((), ())),
        preferred_element_type=jnp.float32,
        precision=jax.lax.Precision.HIGHEST,
    )
    cols = jax.lax.broadcasted_iota(jnp.int32, (_BM, _NP), 1)
    strip_ref[...] = jnp.where(cols < _N, scores, _NEG)
    for k in range(_K):
        s = strip_ref[...]
        m = jnp.max(s, axis=1, keepdims=True)
        hit = s >= m
        idx = jnp.min(jnp.where(hit, cols, _NP), axis=1, keepdims=True)
        vals_ref[:, k:k + 1] = m
        idx_ref[:, k:k + 1] = idx
        strip_ref[...] = jnp.where(cols == idx, _NEG, s)


def _fused_simtopk(h_s, h_t):
    hs_pad = jnp.zeros((_NP, _D), jnp.float32).at[:_N].set(h_s)
    ht_pad = jnp.zeros((_NP, _D), jnp.float32).at[:_N].set(h_t)
    vals, idx = pl.pallas_call(
        _simtopk_body,
        grid=(_NP // _BM,),
        in_specs=[
            pl.BlockSpec((_BM, _D), lambda i: (i, 0)),
            pl.BlockSpec((_NP, _D), lambda i: (0, 0)),
        ],
        out_specs=[
            pl.BlockSpec((_BM, _K), lambda i: (i, 0)),
            pl.BlockSpec((_BM, _K), lambda i: (i, 0)),
        ],
        out_shape=[
            jax.ShapeDtypeStruct((_NP, _K), jnp.float32),
            jax.ShapeDtypeStruct((_NP, _K), jnp.int32),
        ],
        scratch_shapes=[pltpu.VMEM((_BM, _NP), jnp.float32)],
    )(hs_pad, ht_pad)
    return vals[:_N], idx[:_N]


def _seg256_body(x0, x1, srcp, dstp, zbig,
                 out0, out1,
                 acc, src_i, dst_i, rowbuf, rowbuf2, gsem, gsem2, ssem, ssem2):
    c = lax.axis_index("c")
    s = lax.axis_index("s")
    r0 = s * _RT
    pltpu.sync_copy(zbig, acc.at[pl.ds(r0, _RT)])
    plsc.subcore_barrier()

    def gather(j, rb, sem):
        @pl.when(c == 0)
        def _():
            pltpu.async_copy(x0.at[src_i.at[j]], rb, sem)

        @pl.when(c == 1)
        def _():
            pltpu.async_copy(x1.at[src_i.at[j]], rb, sem)

    def gwait(rb, sem):
        @pl.when(c == 0)
        def _():
            pltpu.make_async_copy(x0.at[src_i.at[0]], rb, sem).wait()

        @pl.when(c == 1)
        def _():
            pltpu.make_async_copy(x1.at[src_i.at[0]], rb, sem).wait()

    rbs = (rowbuf, rowbuf2)
    sems = (gsem, gsem2)
    ssems = (ssem, ssem2)

    def swait(j):
        pltpu.make_async_copy(rbs[j % 2], acc.at[dst_i.at[0]],
                              ssems[j % 2]).wait()

    def group(g, carry):
        pltpu.sync_copy(srcp.at[s, pl.ds(g * 8, 8)], src_i)
        pltpu.sync_copy(dstp.at[s, pl.ds(g * 8, 8)], dst_i)
        gather(0, rbs[0], sems[0])
        for j in range(8):
            if j + 1 < 8:
                if j >= 1:
                    swait(j - 1)
                gather(j + 1, rbs[(j + 1) % 2], sems[(j + 1) % 2])
            gwait(rbs[j % 2], sems[j % 2])
            pltpu.async_copy(rbs[j % 2], acc.at[dst_i.at[j]], ssems[j % 2],
                             add=True)
        swait(6)
        swait(7)
        return carry

    lax.fori_loop(0, _CH1 // 8, group, 0)
    plsc.subcore_barrier()

    @pl.when(c == 0)
    def _():
        pltpu.sync_copy(acc.at[pl.ds(r0, _RT)], out0.at[pl.ds(r0, _RT)])

    @pl.when(c == 1)
    def _():
        pltpu.sync_copy(acc.at[pl.ds(r0, _RT)], out1.at[pl.ds(r0, _RT)])


@jax.jit
def _seg256(x, srcp, dstp):
    x0 = x[:, :128]
    x1 = x[:, 128:]
    zbig = jnp.zeros((_RT, 128), jnp.float32)
    mesh = plsc.VectorSubcoreMesh(core_axis_name="c", subcore_axis_name="s")
    out0, out1 = pl.kernel(
        _seg256_body,
        out_type=[
            jax.ShapeDtypeStruct((_NR, 128), jnp.float32),
            jax.ShapeDtypeStruct((_NR, 128), jnp.float32),
        ],
        mesh=mesh,
        scratch_types=[
            pltpu.VMEM_SHARED((_NR, 128), jnp.float32),
            pltpu.VMEM((8, 128), jnp.int32),
            pltpu.VMEM((8, 128), jnp.int32),
            pltpu.VMEM((128, 128), jnp.float32),
            pltpu.VMEM((128, 128), jnp.float32),
            pltpu.SemaphoreType.DMA,
            pltpu.SemaphoreType.DMA,
            pltpu.SemaphoreType.DMA,
            pltpu.SemaphoreType.DMA,
        ],
    )(x0, x1, srcp, dstp, zbig)
    aggb = jnp.concatenate([out0[:_N], out1[:_N]], axis=1)
    return aggb


def _seg32_body(x32, srcp, dstp, z32, out0, out1,
                acc, src_i, dst_i, rowbuf, rowbuf2,
                gsem, gsem2, ssem, ssem2):
    c = lax.axis_index("c")
    s = lax.axis_index("s")
    r0 = s * _RT
    pltpu.sync_copy(z32, acc.at[pl.ds(r0, _RT)])
    plsc.subcore_barrier()
    w = c * _NS + s
    rbs = (rowbuf, rowbuf2)
    sems = (gsem, gsem2)
    ssems = (ssem, ssem2)

    def swait(j):
        pltpu.make_async_copy(rbs[j % 2], acc.at[dst_i.at[0]],
                              ssems[j % 2]).wait()

    def group(g, carry):
        pltpu.sync_copy(srcp.at[w, pl.ds(g * 8, 8)], src_i)
        pltpu.sync_copy(dstp.at[w, pl.ds(g * 8, 8)], dst_i)
        pltpu.async_copy(x32.at[src_i.at[0]], rbs[0], sems[0])
        for j in range(8):
            if j + 1 < 8:
                if j >= 1:
                    swait(j - 1)
                pltpu.async_copy(x32.at[src_i.at[j + 1]], rbs[(j + 1) % 2],
                                 sems[(j + 1) % 2])
            pltpu.make_async_copy(x32.at[src_i.at[0]], rbs[j % 2],
                                  sems[j % 2]).wait()
            pltpu.async_copy(rbs[j % 2], acc.at[dst_i.at[j]], ssems[j % 2],
                             add=True)
        swait(6)
        swait(7)
        return carry

    lax.fori_loop(0, _CH2 // 8, group, 0)
    plsc.subcore_barrier()

    @pl.when(c == 0)
    def _():
        pltpu.sync_copy(acc.at[pl.ds(r0, _RT)], out0.at[pl.ds(r0, _RT)])

    @pl.when(c == 1)
    def _():
        pltpu.sync_copy(acc.at[pl.ds(r0, _RT)], out1.at[pl.ds(r0, _RT)])


@jax.jit
def _seg32(x32, srcp, dstp):
    xp = jnp.zeros((_N, 128), jnp.float32).at[:, :_RD].set(x32)
    z32 = jnp.zeros((_RT, 128), jnp.float32)
    mesh = plsc.VectorSubcoreMesh(core_axis_name="c", subcore_axis_name="s")
    out0, out1 = pl.kernel(
        _seg32_body,
        out_type=[
            jax.ShapeDtypeStruct((_NR, 128), jnp.float32),
            jax.ShapeDtypeStruct((_NR, 128), jnp.float32),
        ],
        mesh=mesh,
        scratch_types=[
            pltpu.VMEM_SHARED((_NR, 128), jnp.float32),
            pltpu.VMEM((8, 128), jnp.int32),
            pltpu.VMEM((8, 128), jnp.int32),
            pltpu.VMEM((128, 128), jnp.float32),
            pltpu.VMEM((128, 128), jnp.float32),
            pltpu.SemaphoreType.DMA,
            pltpu.SemaphoreType.DMA,
            pltpu.SemaphoreType.DMA,
            pltpu.SemaphoreType.DMA,
        ],
    )(xp, srcp, dstp, z32)
    return out0[:_N, :_RD] + out1[:_N, :_RD]


def _pad_edges(edge_index):
    src = edge_index[0]
    dst = edge_index[1]
    pad = _EP - _E
    srcp = jnp.concatenate([src, jnp.zeros((pad,), jnp.int32)])
    dstp = jnp.concatenate([dst, jnp.full((pad,), _N, jnp.int32)])
    return srcp, dstp


def kernel(inputs, x_s, edge_attr_s, x_t, edge_attr_t, W1s, W1n, We1, b1, R,
           W2s, W2n, We2, b2, Wm1, bm1, Wm2, bm2, Ww1, bw1, Ww2, bw2,
           edge_index_s, batch_s, edge_index_t, batch_t,
           index_n1, index_n2, selection_index):
    hi = jax.lax.Precision.HIGHEST

    src_s, dst_s = _pad_edges(edge_index_s)
    src_t, dst_t = _pad_edges(edge_index_t)
    srcp1_s, dstp1_s = src_s.reshape(_NS, _CH1, 128), dst_s.reshape(_NS, _CH1, 128)
    srcp1_t, dstp1_t = src_t.reshape(_NS, _CH1, 128), dst_t.reshape(_NS, _CH1, 128)
    srcp2_s, dstp2_s = src_s.reshape(_NC * _NS, _CH2, 128), dst_s.reshape(_NC * _NS, _CH2, 128)
    srcp2_t, dstp2_t = src_t.reshape(_NC * _NS, _CH2, 128), dst_t.reshape(_NC * _NS, _CH2, 128)
    aggb_s = _seg256(x_s, srcp1_s, dstp1_s)
    aggb_t = _seg256(x_t, srcp1_t, dstp1_t)
    aggs_s = jax.ops.segment_sum(edge_attr_s, edge_index_s[1], num_segments=_N)
    aggs_t = jax.ops.segment_sum(edge_attr_t, edge_index_t[1], num_segments=_N)
    WeW1 = jnp.matmul(We1, W1n, precision=hi)
    h_s = jax.nn.relu(jnp.matmul(x_s, W1s, precision=hi)
                      + jnp.matmul(aggb_s, W1n, precision=hi)
                      + jnp.matmul(aggs_s, WeW1, precision=hi) + b1)
    h_t = jax.nn.relu(jnp.matmul(x_t, W1s, precision=hi)
                      + jnp.matmul(aggb_t, W1n, precision=hi)
                      + jnp.matmul(aggs_t, WeW1, precision=hi) + b1)

    S_val0, S_idx = _fused_simtopk(h_s, h_t)
    S = jax.nn.softmax(S_val0, axis=-1)

    WeW2 = jnp.matmul(We2, W2n, precision=hi)
    aggb_R = _seg32(R, srcp2_t, dstp2_t)
    o_t = jax.nn.relu(jnp.matmul(R, W2s, precision=hi)
                      + jnp.matmul(aggb_R, W2n, precision=hi)
                      + jnp.matmul(aggs_t, WeW2, precision=hi) + b2)

    for _ in range(_NUM_STEPS):
        r_s = jnp.einsum('ik,ikd->id', S, R[S_idx], precision=hi)
        aggb_r = _seg32(r_s, srcp2_s, dstp2_s)
        o_s = jax.nn.relu(jnp.matmul(r_s, W2s, precision=hi)
                          + jnp.matmul(aggb_r, W2n, precision=hi)
                          + jnp.matmul(aggs_s, WeW2, precision=hi) + b2)
        Dm = o_s[:, None, :] - o_t[S_idx]
        d = (jax.nn.relu(jnp.matmul(Dm, Wm1, precision=hi) + bm1) @ Wm2 + bm2)[..., 0]
        S = jax.nn.softmax(S_val0 + d, axis=-1)

    w = jax.nn.sigmoid(jax.nn.relu(inputs @ Ww1 + bw1) @ Ww2 + bw2)
    anchor = jnp.sum(h_s[index_n1] * h_t[index_n2], axis=-1)
    out = S[selection_index] * w[selection_index] + 0.01 * anchor[:, None]
    return out


# speedup vs baseline: 1.2830x; 1.2830x over previous
import functools

import jax
import jax.numpy as jnp
from jax import lax
from jax.experimental import pallas as pl
from jax.experimental.pallas import tpu as pltpu
from jax.experimental.pallas import tpu_sc as plsc

_NUM_STEPS = 2
_K = 10
_N = 10000
_E = 160000
_D = 256
_RD = 32
_NP = 10240
_BM = 256
_NEG = -1e30

_NC, _NS = 2, 16
_CH1 = 80
_CH2 = 40
_EP = _NS * _CH1 * 128
_NR = 10112
_RT = _NR // _NS


def _simtopk_body(hs_ref, ht_ref, vals_ref, idx_ref, strip_ref):
    scores = jax.lax.dot_general(
        hs_ref[...], ht_ref[...],
        (((1,), (1,)), ((), ())),
        preferred_element_type=jnp.float32,
        precision=jax.lax.Precision.HIGHEST,
    )
    cols = jax.lax.broadcasted_iota(jnp.int32, (_BM, _NP), 1)
    strip_ref[...] = jnp.where(cols < _N, scores, _NEG)
    for k in range(_K):
        s = strip_ref[...]
        m = jnp.max(s, axis=1, keepdims=True)
        hit = s >= m
        idx = jnp.min(jnp.where(hit, cols, _NP), axis=1, keepdims=True)
        vals_ref[:, k:k + 1] = m
        idx_ref[:, k:k + 1] = idx
        strip_ref[...] = jnp.where(cols == idx, _NEG, s)


def _fused_simtopk(h_s, h_t):
    hs_pad = jnp.zeros((_NP, _D), jnp.float32).at[:_N].set(h_s)
    ht_pad = jnp.zeros((_NP, _D), jnp.float32).at[:_N].set(h_t)
    vals, idx = pl.pallas_call(
        _simtopk_body,
        grid=(_NP // _BM,),
        in_specs=[
            pl.BlockSpec((_BM, _D), lambda i: (i, 0)),
            pl.BlockSpec((_NP, _D), lambda i: (0, 0)),
        ],
        out_specs=[
            pl.BlockSpec((_BM, _K), lambda i: (i, 0)),
            pl.BlockSpec((_BM, _K), lambda i: (i, 0)),
        ],
        out_shape=[
            jax.ShapeDtypeStruct((_NP, _K), jnp.float32),
            jax.ShapeDtypeStruct((_NP, _K), jnp.int32),
        ],
        scratch_shapes=[pltpu.VMEM((_BM, _NP), jnp.float32)],
    )(hs_pad, ht_pad)
    return vals[:_N], idx[:_N]


def _seg256_body(x0, x1, srcp, dstp, zbig,
                 out0, out1,
                 acc, src_i, dst_i, rowbuf, rowbuf2, gsem, gsem2, ssem, ssem2):
    c = lax.axis_index("c")
    s = lax.axis_index("s")
    r0 = s * _RT
    pltpu.sync_copy(zbig, acc.at[pl.ds(r0, _RT)])
    plsc.subcore_barrier()

    def gather(j, rb, sem):
        @pl.when(c == 0)
        def _():
            pltpu.async_copy(x0.at[src_i.at[j]], rb, sem)

        @pl.when(c == 1)
        def _():
            pltpu.async_copy(x1.at[src_i.at[j]], rb, sem)

    def gwait(rb, sem):
        @pl.when(c == 0)
        def _():
            pltpu.make_async_copy(x0.at[src_i.at[0]], rb, sem).wait()

        @pl.when(c == 1)
        def _():
            pltpu.make_async_copy(x1.at[src_i.at[0]], rb, sem).wait()

    rbs = (rowbuf, rowbuf2)
    sems = (gsem, gsem2)
    ssems = (ssem, ssem2)

    def swait(j):
        pltpu.make_async_copy(rbs[j % 2], acc.at[dst_i.at[0]],
                              ssems[j % 2]).wait()

    def group(g, carry):
        pltpu.sync_copy(srcp.at[s, pl.ds(g * 8, 8)], src_i)
        pltpu.sync_copy(dstp.at[s, pl.ds(g * 8, 8)], dst_i)
        gather(0, rbs[0], sems[0])
        for j in range(8):
            if j + 1 < 8:
                if j >= 1:
                    swait(j - 1)
                gather(j + 1, rbs[(j + 1) % 2], sems[(j + 1) % 2])
            gwait(rbs[j % 2], sems[j % 2])
            pltpu.async_copy(rbs[j % 2], acc.at[dst_i.at[j]], ssems[j % 2],
                             add=True)
        swait(6)
        swait(7)
        return carry

    lax.fori_loop(0, _CH1 // 8, group, 0)
    plsc.subcore_barrier()

    @pl.when(c == 0)
    def _():
        pltpu.sync_copy(acc.at[pl.ds(r0, _RT)], out0.at[pl.ds(r0, _RT)])

    @pl.when(c == 1)
    def _():
        pltpu.sync_copy(acc.at[pl.ds(r0, _RT)], out1.at[pl.ds(r0, _RT)])


@jax.jit
def _seg256(x, srcp, dstp):
    x0 = x[:, :128]
    x1 = x[:, 128:]
    zbig = jnp.zeros((_RT, 128), jnp.float32)
    mesh = plsc.VectorSubcoreMesh(core_axis_name="c", subcore_axis_name="s")
    out0, out1 = pl.kernel(
        _seg256_body,
        out_type=[
            jax.ShapeDtypeStruct((_NR, 128), jnp.float32),
            jax.ShapeDtypeStruct((_NR, 128), jnp.float32),
        ],
        mesh=mesh,
        scratch_types=[
            pltpu.VMEM_SHARED((_NR, 128), jnp.float32),
            pltpu.VMEM((8, 128), jnp.int32),
            pltpu.VMEM((8, 128), jnp.int32),
            pltpu.VMEM((128, 128), jnp.float32),
            pltpu.VMEM((128, 128), jnp.float32),
            pltpu.SemaphoreType.DMA,
            pltpu.SemaphoreType.DMA,
            pltpu.SemaphoreType.DMA,
            pltpu.SemaphoreType.DMA,
        ],
    )(x0, x1, srcp, dstp, zbig)
    aggb = jnp.concatenate([out0[:_N], out1[:_N]], axis=1)
    return aggb


def _seg32_body(x32, srcp, dstp, z32, out0, out1,
                acc, src_i, dst_i, rowbuf, rowbuf2,
                gsem, gsem2, ssem, ssem2):
    c = lax.axis_index("c")
    s = lax.axis_index("s")
    r0 = s * _RT
    pltpu.sync_copy(z32, acc.at[pl.ds(r0, _RT)])
    plsc.subcore_barrier()
    w = c * _NS + s
    rbs = (rowbuf, rowbuf2)
    sems = (gsem, gsem2)
    ssems = (ssem, ssem2)

    def swait(j):
        pltpu.make_async_copy(rbs[j % 2], acc.at[dst_i.at[0]],
                              ssems[j % 2]).wait()

    def group(g, carry):
        pltpu.sync_copy(srcp.at[w, pl.ds(g * 8, 8)], src_i)
        pltpu.sync_copy(dstp.at[w, pl.ds(g * 8, 8)], dst_i)
        pltpu.async_copy(x32.at[src_i.at[0]], rbs[0], sems[0])
        for j in range(8):
            if j + 1 < 8:
                if j >= 1:
                    swait(j - 1)
                pltpu.async_copy(x32.at[src_i.at[j + 1]], rbs[(j + 1) % 2],
                                 sems[(j + 1) % 2])
            pltpu.make_async_copy(x32.at[src_i.at[0]], rbs[j % 2],
                                  sems[j % 2]).wait()
            pltpu.async_copy(rbs[j % 2], acc.at[dst_i.at[j]], ssems[j % 2],
                             add=True)
        swait(6)
        swait(7)
        return carry

    lax.fori_loop(0, _CH2 // 8, group, 0)
    plsc.subcore_barrier()

    @pl.when(c == 0)
    def _():
        pltpu.sync_copy(acc.at[pl.ds(r0, _RT)], out0.at[pl.ds(r0, _RT)])

    @pl.when(c == 1)
    def _():
        pltpu.sync_copy(acc.at[pl.ds(r0, _RT)], out1.at[pl.ds(r0, _RT)])


@jax.jit
def _seg32(x32, srcp, dstp):
    xp = jnp.zeros((_N, 128), jnp.float32).at[:, :_RD].set(x32)
    z32 = jnp.zeros((_RT, 128), jnp.float32)
    mesh = plsc.VectorSubcoreMesh(core_axis_name="c", subcore_axis_name="s")
    out0, out1 = pl.kernel(
        _seg32_body,
        out_type=[
            jax.ShapeDtypeStruct((_NR, 128), jnp.float32),
            jax.ShapeDtypeStruct((_NR, 128), jnp.float32),
        ],
        mesh=mesh,
        scratch_types=[
            pltpu.VMEM_SHARED((_NR, 128), jnp.float32),
            pltpu.VMEM((8, 128), jnp.int32),
            pltpu.VMEM((8, 128), jnp.int32),
            pltpu.VMEM((128, 128), jnp.float32),
            pltpu.VMEM((128, 128), jnp.float32),
            pltpu.SemaphoreType.DMA,
            pltpu.SemaphoreType.DMA,
            pltpu.SemaphoreType.DMA,
            pltpu.SemaphoreType.DMA,
        ],
    )(xp, srcp, dstp, z32)
    return out0[:_N, :_RD] + out1[:_N, :_RD]


def _sege_body(ea4, dstp, out, acc, dbuf, eb0, eb1, eb2, eb3):
    c = lax.axis_index("c")
    s = lax.axis_index("s")
    w = c * _NS + s

    def zero(j, carry):
        acc[pl.ds(j * 16, 16)] = jnp.zeros((16,), jnp.float32)
        return carry

    lax.fori_loop(0, (_NR * 4) // 16, zero, 0)
    pltpu.sync_copy(dstp.at[w], dbuf)
    ebs = (eb0, eb1, eb2, eb3)
    for cc in range(4):
        pltpu.sync_copy(ea4.at[cc, pl.ds(w * (_CH2 * 128), _CH2 * 128)],
                        ebs[cc])

    def step(t, carry):
        didx = dbuf[pl.ds(t * 16, 16)] * 4
        for cc in range(4):
            v = ebs[cc][pl.ds(t * 16, 16)]
            plsc.addupdate_scatter(acc, [didx + cc], v)
        return carry

    lax.fori_loop(0, (_CH2 * 128) // 16, step, 0)
    pltpu.sync_copy(acc, out.at[w])


@jax.jit
def _sege(edge_attr, dstf):
    ea4 = jnp.zeros((4, _EP), jnp.float32).at[:, :_E].set(edge_attr.T)
    mesh = plsc.VectorSubcoreMesh(core_axis_name="c", subcore_axis_name="s")
    out = pl.kernel(
        _sege_body,
        out_type=jax.ShapeDtypeStruct((_NC * _NS, _NR * 4), jnp.float32),
        mesh=mesh,
        scratch_types=[
            pltpu.VMEM((_NR * 4,), jnp.float32),
            pltpu.VMEM((_CH2 * 128,), jnp.int32),
            pltpu.VMEM((_CH2 * 128,), jnp.float32),
            pltpu.VMEM((_CH2 * 128,), jnp.float32),
            pltpu.VMEM((_CH2 * 128,), jnp.float32),
            pltpu.VMEM((_CH2 * 128,), jnp.float32),
        ],
        compiler_params=pltpu.CompilerParams(needs_layout_passes=False),
    )(ea4, dstf)
    return out.sum(axis=0).reshape(_NR, 4)[:_N]


def _pad_edges(edge_index):
    src = edge_index[0]
    dst = edge_index[1]
    pad = _EP - _E
    srcp = jnp.concatenate([src, jnp.zeros((pad,), jnp.int32)])
    dstp = jnp.concatenate([dst, jnp.full((pad,), _N, jnp.int32)])
    return srcp, dstp


def kernel(inputs, x_s, edge_attr_s, x_t, edge_attr_t, W1s, W1n, We1, b1, R,
           W2s, W2n, We2, b2, Wm1, bm1, Wm2, bm2, Ww1, bw1, Ww2, bw2,
           edge_index_s, batch_s, edge_index_t, batch_t,
           index_n1, index_n2, selection_index):
    hi = jax.lax.Precision.HIGHEST

    src_s, dst_s = _pad_edges(edge_index_s)
    src_t, dst_t = _pad_edges(edge_index_t)
    srcp1_s, dstp1_s = src_s.reshape(_NS, _CH1, 128), dst_s.reshape(_NS, _CH1, 128)
    srcp1_t, dstp1_t = src_t.reshape(_NS, _CH1, 128), dst_t.reshape(_NS, _CH1, 128)
    srcp2_s, dstp2_s = src_s.reshape(_NC * _NS, _CH2, 128), dst_s.reshape(_NC * _NS, _CH2, 128)
    srcp2_t, dstp2_t = src_t.reshape(_NC * _NS, _CH2, 128), dst_t.reshape(_NC * _NS, _CH2, 128)
    aggb_s = _seg256(x_s, srcp1_s, dstp1_s)
    aggb_t = _seg256(x_t, srcp1_t, dstp1_t)
    aggs_s = _sege(edge_attr_s, dst_s.reshape(_NC * _NS, _CH2 * 128))
    aggs_t = _sege(edge_attr_t, dst_t.reshape(_NC * _NS, _CH2 * 128))
    WeW1 = jnp.matmul(We1, W1n, precision=hi)
    h_s = jax.nn.relu(jnp.matmul(x_s, W1s, precision=hi)
                      + jnp.matmul(aggb_s, W1n, precision=hi)
                      + jnp.matmul(aggs_s, WeW1, precision=hi) + b1)
    h_t = jax.nn.relu(jnp.matmul(x_t, W1s, precision=hi)
                      + jnp.matmul(aggb_t, W1n, precision=hi)
                      + jnp.matmul(aggs_t, WeW1, precision=hi) + b1)

    S_val0, S_idx = _fused_simtopk(h_s, h_t)
    S = jax.nn.softmax(S_val0, axis=-1)

    WeW2 = jnp.matmul(We2, W2n, precision=hi)
    aggb_R = _seg32(R, srcp2_t, dstp2_t)
    o_t = jax.nn.relu(jnp.matmul(R, W2s, precision=hi)
                      + jnp.matmul(aggb_R, W2n, precision=hi)
                      + jnp.matmul(aggs_t, WeW2, precision=hi) + b2)

    for _ in range(_NUM_STEPS):
        r_s = jnp.einsum('ik,ikd->id', S, R[S_idx], precision=hi)
        aggb_r = _seg32(r_s, srcp2_s, dstp2_s)
        o_s = jax.nn.relu(jnp.matmul(r_s, W2s, precision=hi)
                          + jnp.matmul(aggb_r, W2n, precision=hi)
                          + jnp.matmul(aggs_s, WeW2, precision=hi) + b2)
        Dm = o_s[:, None, :] - o_t[S_idx]
        d = (jax.nn.relu(jnp.matmul(Dm, Wm1, precision=hi) + bm1) @ Wm2 + bm2)[..., 0]
        S = jax.nn.softmax(S_val0 + d, axis=-1)

    w = jax.nn.sigmoid(jax.nn.relu(inputs @ Ww1 + bw1) @ Ww2 + bw2)
    anchor = jnp.sum(h_s[index_n1] * h_t[index_n2], axis=-1)
    out = S[selection_index] * w[selection_index] + 0.01 * anchor[:, None]
    return out
